# Initial kernel scaffold; baseline (speedup 1.0000x reference)
import jax
import jax.numpy as jnp
from jax.experimental import pallas as pl


def kernel(x, Wg, W1, b1, W2, b2):
    raise NotImplementedError("write your pallas kernel here")


# speedup vs baseline: 1.8246x; 1.8246x over previous
import jax
import jax.numpy as jnp
from jax.experimental import pallas as pl
from jax.experimental.pallas import tpu as pltpu

E = 8
K = 2
D_MODEL = 1024
D_FF = 4096
T = 4096
C = 1024

FT = 1024
NF = D_FF // FT
TC = 1024
TT = 512


def _incl_cumsum_rows(a):
    n = a.shape[0]
    d = 1
    while d < n:
        pad = jnp.zeros((d, a.shape[1]), a.dtype)
        a = a + jnp.concatenate([pad, a[:-d]], axis=0)
        d *= 2
    return a


def _routing_kernel(x_ref, wg_ref, slot0_ref, slot1_ref, w0_ref, w1_ref):
    x = x_ref[...]
    wg = wg_ref[...]
    logits = jax.lax.dot_general(
        x, wg, (((1,), (0,)), ((), ())),
        preferred_element_type=jnp.float32)
    m = jnp.max(logits, axis=1, keepdims=True)
    p = jnp.exp(logits - m)
    gates = p / jnp.sum(p, axis=1, keepdims=True)

    lane = jax.lax.broadcasted_iota(jnp.int32, (T, E), 1)
    v0 = jnp.max(gates, axis=1, keepdims=True)
    i0 = jnp.min(jnp.where(gates == v0, lane, E), axis=1, keepdims=True)
    g1 = jnp.where(lane == i0, -1.0, gates)
    v1 = jnp.max(g1, axis=1, keepdims=True)
    i1 = jnp.min(jnp.where(g1 == v1, lane, E), axis=1, keepdims=True)
    denom = v0 + v1 + 1e-9
    w0_ref[...] = v0 / denom
    w1_ref[...] = v1 / denom

    oh0 = (lane == i0).astype(jnp.float32)
    oh1 = (lane == i1).astype(jnp.float32)
    cs0 = _incl_cumsum_rows(oh0)
    cs1 = _incl_cumsum_rows(oh1)
    pos0 = jnp.sum((cs0 - oh0) * oh0, axis=1, keepdims=True)
    off = jnp.sum(oh0, axis=0, keepdims=True)
    pos1 = (jnp.sum((cs1 - oh1) * oh1, axis=1, keepdims=True)
            + jnp.sum(off * oh1, axis=1, keepdims=True))
    pos0 = pos0.astype(jnp.int32)
    pos1 = pos1.astype(jnp.int32)
    slot0 = i0 * C + pos0
    slot1 = i1 * C + pos1
    slot0_ref[...] = jnp.where(pos0 < C, slot0, -1)
    slot1_ref[...] = jnp.where(pos1 < C, slot1, -1)


def _dispatch_kernel(slot0_ref, slot1_ref, x_ref, disp_ref):
    e = pl.program_id(0)
    c = pl.program_id(1)
    s_row = jax.lax.broadcasted_iota(jnp.int32, (TC, C), 1) + e * C
    q = ((slot0_ref[...] == s_row) | (slot1_ref[...] == s_row)
         ).astype(jnp.float32)
    d = jax.lax.dot_general(
        q, x_ref[...], (((0,), (0,)), ((), ())),
        preferred_element_type=jnp.float32)

    @pl.when(c == 0)
    def _init():
        disp_ref[0] = d

    @pl.when(c > 0)
    def _acc():
        disp_ref[0] += d


def _ffn_kernel(disp_ref, w1_ref, b1_ref, w2_ref, b2_ref, out_ref, acc_ref):
    f = pl.program_id(1)

    h = jax.lax.dot_general(
        disp_ref[0], w1_ref[0], (((1,), (0,)), ((), ())),
        preferred_element_type=jnp.float32)
    h = h + b1_ref[0]
    h = 0.5 * h * (1.0 + jax.lax.erf(h * 0.7071067811865476))
    d2 = jax.lax.dot_general(
        h, w2_ref[0], (((1,), (0,)), ((), ())),
        preferred_element_type=jnp.float32)

    @pl.when(f == 0)
    def _init():
        acc_ref[...] = d2

    @pl.when(f > 0)
    def _acc():
        acc_ref[...] += d2

    @pl.when(f == NF - 1)
    def _finish():
        out_ref[0] = acc_ref[...] + b2_ref[0]


def _combine_kernel(slot0_ref, slot1_ref, w0_ref, w1_ref, y_ref, out_ref,
                    acc_ref):
    s = pl.program_id(1)

    @pl.when(s == 0)
    def _init():
        acc_ref[...] = jnp.zeros_like(acc_ref)

    s_row = jax.lax.broadcasted_iota(jnp.int32, (TT, C), 1) + s * C
    g = (jnp.where(slot0_ref[...] == s_row, w0_ref[...], 0.0)
         + jnp.where(slot1_ref[...] == s_row, w1_ref[...], 0.0))
    acc_ref[...] += jax.lax.dot_general(
        g, y_ref[0], (((1,), (0,)), ((), ())),
        preferred_element_type=jnp.float32)

    @pl.when(s == E - 1)
    def _finish():
        out_ref[...] = acc_ref[...]


def kernel(x, Wg, W1, b1, W2, b2):
    slot0, slot1, w0, w1 = pl.pallas_call(
        _routing_kernel,
        out_shape=(
            jax.ShapeDtypeStruct((T, 1), jnp.int32),
            jax.ShapeDtypeStruct((T, 1), jnp.int32),
            jax.ShapeDtypeStruct((T, 1), jnp.float32),
            jax.ShapeDtypeStruct((T, 1), jnp.float32),
        ),
    )(x, Wg)

    disp = pl.pallas_call(
        _dispatch_kernel,
        grid=(E, T // TC),
        in_specs=[
            pl.BlockSpec((TC, 1), lambda e, c: (c, 0)),
            pl.BlockSpec((TC, 1), lambda e, c: (c, 0)),
            pl.BlockSpec((TC, D_MODEL), lambda e, c: (c, 0)),
        ],
        out_specs=pl.BlockSpec((1, C, D_MODEL), lambda e, c: (e, 0, 0)),
        out_shape=jax.ShapeDtypeStruct((E, C, D_MODEL), jnp.float32),
    )(slot0, slot1, x)

    y = pl.pallas_call(
        _ffn_kernel,
        grid=(E, NF),
        in_specs=[
            pl.BlockSpec((1, C, D_MODEL), lambda e, f: (e, 0, 0)),
            pl.BlockSpec((1, D_MODEL, FT), lambda e, f: (e, 0, f)),
            pl.BlockSpec((1, 1, FT), lambda e, f: (e, 0, f)),
            pl.BlockSpec((1, FT, D_MODEL), lambda e, f: (e, f, 0)),
            pl.BlockSpec((1, 1, D_MODEL), lambda e, f: (e, 0, 0)),
        ],
        out_specs=pl.BlockSpec((1, C, D_MODEL), lambda e, f: (e, 0, 0)),
        out_shape=jax.ShapeDtypeStruct((E, C, D_MODEL), jnp.float32),
        scratch_shapes=[pltpu.VMEM((C, D_MODEL), jnp.float32)],
    )(disp, W1, b1.reshape(E, 1, D_FF), W2, b2.reshape(E, 1, D_MODEL))

    y_flat = y.reshape(E * C, D_MODEL)
    out = pl.pallas_call(
        _combine_kernel,
        grid=(T // TT, E),
        in_specs=[
            pl.BlockSpec((TT, 1), lambda t, s: (t, 0)),
            pl.BlockSpec((TT, 1), lambda t, s: (t, 0)),
            pl.BlockSpec((TT, 1), lambda t, s: (t, 0)),
            pl.BlockSpec((TT, 1), lambda t, s: (t, 0)),
            pl.BlockSpec((1, C, D_MODEL), lambda t, s: (s, 0, 0)),
        ],
        out_specs=pl.BlockSpec((TT, D_MODEL), lambda t, s: (t, 0)),
        out_shape=jax.ShapeDtypeStruct((T, D_MODEL), jnp.float32),
        scratch_shapes=[pltpu.VMEM((TT, D_MODEL), jnp.float32)],
    )(slot0, slot1, w0, w1, y_flat.reshape(E, C, D_MODEL))

    return out
